# Initial kernel scaffold; baseline (speedup 1.0000x reference)
import jax
import jax.numpy as jnp
from jax.experimental import pallas as pl


def kernel(X_prefix, edge_index, Wxz0, Wxz1, bxz, Whz0, Whz1, bhz, Wxr0, Wxr1, bxr, Whr0, Whr1, bhr, Wxh0, Wxh1, bxh, Whh0, Whh1, bhh, L1W, L1b, L2W, L2b):
    raise NotImplementedError("write your pallas kernel here")


# speedup vs baseline: 1.9997x; 1.9997x over previous
import jax
import jax.numpy as jnp
from jax import lax
from jax.experimental import pallas as pl
from jax.experimental.pallas import tpu as pltpu
from jax.experimental.pallas import tpu_sc as plsc

NN = 10000
DD = 256
EE = 160000
NSUB = 16
NCORE = 2
NW = NSUB * NCORE
NPT = 313
NNP = NW * NPT
ACCR = 320
CH = 128
EPAD = EE + NW * CH
RB = 1000
NB = NN // RB


def _sc_mesh():
    return plsc.VectorSubcoreMesh(core_axis_name="c", subcore_axis_name="s")


def _spmm(y, srcp, dstlp, meta, zacc):
    def body(y_hbm, srcp_hbm, dstlp_hbm, meta_hbm, z_hbm, out_hbm,
             idx_v, dst_v, rows_v, acc_v, meta_v, sem):
        w = lax.axis_index("c") * NSUB + lax.axis_index("s")
        pltpu.sync_copy(meta_hbm, meta_v)
        pltpu.sync_copy(z_hbm, acc_v)
        off_w = meta_v[pl.ds(w, 16)][0]
        nch_w = meta_v[pl.ds(32 + w, 16)][0]

        def chunk(i, carry):
            off = (off_w + i) * CH
            pltpu.sync_copy(srcp_hbm.at[pl.ds(off, CH)], idx_v)
            pltpu.sync_copy(dstlp_hbm.at[pl.ds(off, CH)],
                            dst_v.at[pl.ds(0, CH)])
            pltpu.async_copy(y_hbm.at[idx_v], rows_v, sem).wait()

            def edge(e, c2):
                base = dst_v[pl.ds(e, 16)][0] * DD
                for j in range(DD // 16):
                    v = rows_v[e, pl.ds(j * 16, 16)]
                    plsc.addupdate(acc_v.at[pl.ds(base + j * 16, 16)], v)
                return c2

            lax.fori_loop(0, CH, edge, 0)
            return carry

        lax.fori_loop(0, nch_w, chunk, 0)
        pltpu.sync_copy(acc_v.at[pl.ds(0, NPT * DD)],
                        out_hbm.at[pl.ds(w * NPT * DD, NPT * DD)])

    f = pl.kernel(
        body,
        out_type=jax.ShapeDtypeStruct((NNP * DD,), jnp.float32),
        mesh=_sc_mesh(),
        scratch_types=[
            pltpu.VMEM((CH,), jnp.int32),
            pltpu.VMEM((CH + 16,), jnp.int32),
            pltpu.VMEM((CH, DD), jnp.float32),
            pltpu.VMEM((ACCR * DD,), jnp.float32),
            pltpu.VMEM((80,), jnp.int32),
            pltpu.SemaphoreType.DMA,
        ],
    )
    return f(y, srcp, dstlp, meta, zacc).reshape(NNP, DD)


def _degree16(srclp, meta, z16):
    def body(srclp_hbm, meta_hbm, z_hbm, out_hbm, idx_v, acc_v, meta_v):
        w = lax.axis_index("c") * NSUB + lax.axis_index("s")
        pltpu.sync_copy(meta_hbm, meta_v)
        pltpu.sync_copy(z_hbm, acc_v)
        off_w = meta_v[pl.ds(w, 16)][0]
        nch_w = meta_v[pl.ds(32 + w, 16)][0]
        ones = jnp.ones((16,), jnp.float32)

        def chunk(i, carry):
            off = (off_w + i) * CH
            pltpu.sync_copy(srclp_hbm.at[pl.ds(off, CH)],
                            idx_v.at[pl.ds(0, CH)])

            def edge(e, c2):
                base = idx_v[pl.ds(e, 16)][0] * 16
                plsc.addupdate(acc_v.at[pl.ds(base, 16)], ones)
                return c2

            lax.fori_loop(0, CH, edge, 0)
            return carry

        lax.fori_loop(0, nch_w, chunk, 0)
        pltpu.sync_copy(acc_v.at[pl.ds(0, NPT * 16)],
                        out_hbm.at[pl.ds(w * NPT * 16, NPT * 16)])

    f = pl.kernel(
        body,
        out_type=jax.ShapeDtypeStruct((NNP * 16,), jnp.float32),
        mesh=_sc_mesh(),
        scratch_types=[
            pltpu.VMEM((CH + 16,), jnp.int32),
            pltpu.VMEM((ACCR * 16,), jnp.float32),
            pltpu.VMEM((80,), jnp.int32),
        ],
    )
    return f(srclp, meta, z16).reshape(NNP, 16)


def _prep(deg16, x_all):
    tt = x_all.shape[0]

    def body(deg_ref, x_ref, dis_ref, y_ref):
        deg = jnp.sum(deg_ref[...], axis=1, keepdims=True) * (1.0 / 16.0)
        dis = jnp.where(deg > 0.0, lax.rsqrt(jnp.maximum(deg, 1e-12)), 0.0)
        dis_ref[...] = dis
        y_ref[0] = x_ref[0] * dis

    return pl.pallas_call(
        body,
        grid=(tt, NB),
        in_specs=[
            pl.BlockSpec((RB, 16), lambda t, i: (i, 0)),
            pl.BlockSpec((1, RB, DD), lambda t, i: (t, i, 0)),
        ],
        out_specs=[
            pl.BlockSpec((RB, 1), lambda t, i: (i, 0)),
            pl.BlockSpec((1, RB, DD), lambda t, i: (t, i, 0)),
        ],
        out_shape=[
            jax.ShapeDtypeStruct((NN, 1), jnp.float32),
            jax.ShapeDtypeStruct((tt, NN, DD), jnp.float32),
        ],
    )(deg16, x_all)


def _pmat(x, accx, dis, w0c, w1c, bc):
    def body(x_ref, a_ref, d_ref, w0_ref, w1_ref, b_ref, o_ref):
        agg = -(d_ref[...] * a_ref[...])
        p = jnp.dot(x_ref[...], w0_ref[...], preferred_element_type=jnp.float32)
        p += jnp.dot(agg, w1_ref[...], preferred_element_type=jnp.float32)
        o_ref[...] = p + b_ref[...]

    return pl.pallas_call(
        body,
        grid=(NB,),
        in_specs=[
            pl.BlockSpec((RB, DD), lambda i: (i, 0)),
            pl.BlockSpec((RB, DD), lambda i: (i, 0)),
            pl.BlockSpec((RB, 1), lambda i: (i, 0)),
            pl.BlockSpec((DD, 3 * DD), lambda i: (0, 0)),
            pl.BlockSpec((DD, 3 * DD), lambda i: (0, 0)),
            pl.BlockSpec((1, 3 * DD), lambda i: (0, 0)),
        ],
        out_specs=pl.BlockSpec((RB, 3 * DD), lambda i: (i, 0)),
        out_shape=jax.ShapeDtypeStruct((NN, 3 * DD), jnp.float32),
    )(x, accx, dis, w0c, w1c, bc)


def _step0(p0):
    def body(p_ref, h_ref):
        p = p_ref[...]
        z = jax.nn.sigmoid(p[:, :DD])
        ht = jnp.tanh(p[:, 2 * DD:])
        h_ref[...] = (1.0 - z) * ht

    return pl.pallas_call(
        body,
        grid=(NB,),
        in_specs=[pl.BlockSpec((RB, 3 * DD), lambda i: (i, 0))],
        out_specs=pl.BlockSpec((RB, DD), lambda i: (i, 0)),
        out_shape=jax.ShapeDtypeStruct((NN, DD), jnp.float32),
    )(p0)


def _scale(h, dis):
    def body(h_ref, d_ref, y_ref):
        y_ref[...] = h_ref[...] * d_ref[...]

    return pl.pallas_call(
        body,
        grid=(NB,),
        in_specs=[
            pl.BlockSpec((RB, DD), lambda i: (i, 0)),
            pl.BlockSpec((RB, 1), lambda i: (i, 0)),
        ],
        out_specs=pl.BlockSpec((RB, DD), lambda i: (i, 0)),
        out_shape=jax.ShapeDtypeStruct((NN, DD), jnp.float32),
    )(h, dis)


def _gates(p, h, acch, dis, wh0c, wh1c):
    def body(p_ref, h_ref, a_ref, d_ref, w0_ref, w1_ref,
             z_ref, hr_ref, yhr_ref):
        h = h_ref[...]
        dis = d_ref[...]
        agg = -(dis * a_ref[...])
        s = jnp.dot(h, w0_ref[...], preferred_element_type=jnp.float32)
        s += jnp.dot(agg, w1_ref[...], preferred_element_type=jnp.float32)
        s = jax.nn.sigmoid(p_ref[...] + s)
        z = s[:, :DD]
        r = s[:, DD:]
        hr = h * r
        z_ref[...] = z
        hr_ref[...] = hr
        yhr_ref[...] = dis * hr

    return pl.pallas_call(
        body,
        grid=(NB,),
        in_specs=[
            pl.BlockSpec((RB, 2 * DD), lambda i: (i, 0)),
            pl.BlockSpec((RB, DD), lambda i: (i, 0)),
            pl.BlockSpec((RB, DD), lambda i: (i, 0)),
            pl.BlockSpec((RB, 1), lambda i: (i, 0)),
            pl.BlockSpec((DD, 2 * DD), lambda i: (0, 0)),
            pl.BlockSpec((DD, 2 * DD), lambda i: (0, 0)),
        ],
        out_specs=[
            pl.BlockSpec((RB, DD), lambda i: (i, 0)),
            pl.BlockSpec((RB, DD), lambda i: (i, 0)),
            pl.BlockSpec((RB, DD), lambda i: (i, 0)),
        ],
        out_shape=[
            jax.ShapeDtypeStruct((NN, DD), jnp.float32),
            jax.ShapeDtypeStruct((NN, DD), jnp.float32),
            jax.ShapeDtypeStruct((NN, DD), jnp.float32),
        ],
    )(p, h, acch, dis, wh0c, wh1c)


def _update(ph, hr, acchr, dis, whh0, whh1, z, h):
    def body(p_ref, hr_ref, a_ref, d_ref, w0_ref, w1_ref, z_ref, h_ref,
             o_ref):
        agg = -(d_ref[...] * a_ref[...])
        s = jnp.dot(hr_ref[...], w0_ref[...],
                    preferred_element_type=jnp.float32)
        s += jnp.dot(agg, w1_ref[...], preferred_element_type=jnp.float32)
        ht = jnp.tanh(p_ref[...] + s)
        z = z_ref[...]
        o_ref[...] = z * h_ref[...] + (1.0 - z) * ht

    return pl.pallas_call(
        body,
        grid=(NB,),
        in_specs=[
            pl.BlockSpec((RB, DD), lambda i: (i, 2)),
            pl.BlockSpec((RB, DD), lambda i: (i, 0)),
            pl.BlockSpec((RB, DD), lambda i: (i, 0)),
            pl.BlockSpec((RB, 1), lambda i: (i, 0)),
            pl.BlockSpec((DD, DD), lambda i: (0, 0)),
            pl.BlockSpec((DD, DD), lambda i: (0, 0)),
            pl.BlockSpec((RB, DD), lambda i: (i, 0)),
            pl.BlockSpec((RB, DD), lambda i: (i, 0)),
        ],
        out_specs=pl.BlockSpec((RB, DD), lambda i: (i, 0)),
        out_shape=jax.ShapeDtypeStruct((NN, DD), jnp.float32),
    )(ph, hr, acchr, dis, whh0, whh1, z, h)


def _head(h, l1w, l1b, l2wt, l2b):
    def body(h_ref, w1_ref, b1_ref, w2_ref, b2_ref, o_ref):
        g = jnp.sum(h_ref[...], axis=0, keepdims=True) * (1.0 / NN)
        h1 = jax.nn.relu(
            jnp.dot(g, w1_ref[...], preferred_element_type=jnp.float32)
            + b1_ref[...])
        o_ref[...] = jnp.sum(h1 * w2_ref[...], axis=1,
                             keepdims=True) + b2_ref[...]

    return pl.pallas_call(
        body,
        grid=(1,),
        in_specs=[
            pl.BlockSpec((NN, DD), lambda i: (0, 0)),
            pl.BlockSpec((DD, DD), lambda i: (0, 0)),
            pl.BlockSpec((1, DD), lambda i: (0, 0)),
            pl.BlockSpec((1, DD), lambda i: (0, 0)),
            pl.BlockSpec((1, 1), lambda i: (0, 0)),
        ],
        out_specs=pl.BlockSpec((1, 1), lambda i: (0, 0)),
        out_shape=jax.ShapeDtypeStruct((1, 1), jnp.float32),
    )(h, l1w, l1b, l2wt, l2b)


def _partition32(key_idx, val_idx):
    order = jnp.argsort(key_idx)
    ks = key_idx[order]
    vs = val_idx[order]
    owner = ks // NPT
    cnt = jnp.zeros((NW,), jnp.int32).at[owner].add(1)
    padcnt = ((cnt + CH - 1) // CH) * CH
    zero1 = jnp.zeros((1,), jnp.int32)
    off = jnp.concatenate([zero1, jnp.cumsum(padcnt)[:-1]])
    start = jnp.concatenate([zero1, jnp.cumsum(cnt)[:-1]])
    pos = off[owner] + jnp.arange(EE, dtype=jnp.int32) - start[owner]
    varr = jnp.zeros((EPAD,), jnp.int32).at[pos].set(vs)
    karr = jnp.full((EPAD,), NPT, jnp.int32).at[pos].set(ks - owner * NPT)
    meta = jnp.zeros((80,), jnp.int32)
    meta = meta.at[jnp.arange(NW)].set(off // CH)
    meta = meta.at[NW + jnp.arange(NW)].set(padcnt // CH)
    return varr, karr, meta


def kernel(X_prefix, edge_index, Wxz0, Wxz1, bxz, Whz0, Whz1, bhz,
           Wxr0, Wxr1, bxr, Whr0, Whr1, bhr, Wxh0, Wxh1, bxh,
           Whh0, Whh1, bhh, L1W, L1b, L2W, L2b):
    src = edge_index[0]
    dst = edge_index[1]
    tt = X_prefix.shape[0]

    srcp, dstlp, meta_d = _partition32(dst, src)
    _, srclp, meta_s = _partition32(src, src)

    zacc = jnp.zeros((ACCR * DD,), jnp.float32)
    z16 = jnp.zeros((ACCR * 16,), jnp.float32)

    deg16 = _degree16(srclp, meta_s, z16)
    dis, y_all = _prep(deg16, X_prefix)

    w0c = jnp.concatenate([Wxz0, Wxr0, Wxh0], axis=1)
    w1c = jnp.concatenate([Wxz1, Wxr1, Wxh1], axis=1)
    bc = jnp.reshape(jnp.concatenate([bxz + bhz, bxr + bhr, bxh + bhh]),
                     (1, 3 * DD))
    p_list = []
    for t in range(tt):
        accx = _spmm(y_all[t], srcp, dstlp, meta_d, zacc)
        p_list.append(_pmat(X_prefix[t], accx, dis, w0c, w1c, bc))

    wh0c = jnp.concatenate([Whz0, Whr0], axis=1)
    wh1c = jnp.concatenate([Whz1, Whr1], axis=1)
    h = _step0(p_list[0])
    for t in range(1, tt):
        yh = _scale(h, dis)
        acch = _spmm(yh, srcp, dstlp, meta_d, zacc)
        z, hr, yhr = _gates(p_list[t][:, :2 * DD], h, acch, dis, wh0c, wh1c)
        acchr = _spmm(yhr, srcp, dstlp, meta_d, zacc)
        h = _update(p_list[t], hr, acchr, dis, Whh0, Whh1, z, h)

    out = _head(h, L1W, jnp.reshape(L1b, (1, DD)),
                jnp.reshape(L2W, (1, DD)), jnp.reshape(L2b, (1, 1)))
    return jnp.reshape(out, ())


# speedup vs baseline: 2.1457x; 1.0730x over previous
import jax
import jax.numpy as jnp
from jax import lax
from jax.experimental import pallas as pl
from jax.experimental.pallas import tpu as pltpu
from jax.experimental.pallas import tpu_sc as plsc

NN = 10000
DD = 256
EE = 160000
NSUB = 16
NCORE = 2
NW = NSUB * NCORE
NPT = 313
NNP = NW * NPT
ACCR = 320
CH = 128
EPAD = EE + NW * CH
RB = 1000
NB = NN // RB


def _sc_mesh():
    return plsc.VectorSubcoreMesh(core_axis_name="c", subcore_axis_name="s")


def _spmm(y, srcp, dstlp, meta, zacc):
    def body(y_hbm, srcp_hbm, dstlp_hbm, meta_hbm, z_hbm, out_hbm,
             idx_v, dst_v, rows_v, acc_v, meta_v, sem):
        w = lax.axis_index("c") * NSUB + lax.axis_index("s")
        pltpu.sync_copy(meta_hbm, meta_v)
        pltpu.sync_copy(z_hbm, acc_v)
        off_w = meta_v[pl.ds(w, 16)][0]
        nch_w = meta_v[pl.ds(32 + w, 16)][0]

        def chunk(i, carry):
            off = (off_w + i) * CH
            pltpu.sync_copy(srcp_hbm.at[pl.ds(off, CH)], idx_v)
            pltpu.sync_copy(dstlp_hbm.at[pl.ds(off, CH)],
                            dst_v.at[pl.ds(0, CH)])
            pltpu.async_copy(y_hbm.at[idx_v], rows_v, sem).wait()

            def group(g, c2):
                d16 = dst_v[pl.ds(g * 16, 16)]
                for e in range(16):
                    base = d16[e] * DD
                    row = g * 16 + e
                    for j in range(DD // 16):
                        v = rows_v[row, pl.ds(j * 16, 16)]
                        plsc.addupdate(acc_v.at[pl.ds(base + j * 16, 16)], v)
                return c2

            lax.fori_loop(0, CH // 16, group, 0)
            return carry

        lax.fori_loop(0, nch_w, chunk, 0)
        pltpu.sync_copy(acc_v.at[pl.ds(0, NPT * DD)],
                        out_hbm.at[pl.ds(w * NPT * DD, NPT * DD)])

    f = pl.kernel(
        body,
        out_type=jax.ShapeDtypeStruct((NNP * DD,), jnp.float32),
        mesh=_sc_mesh(),
        scratch_types=[
            pltpu.VMEM((CH,), jnp.int32),
            pltpu.VMEM((CH + 16,), jnp.int32),
            pltpu.VMEM((CH, DD), jnp.float32),
            pltpu.VMEM((ACCR * DD,), jnp.float32),
            pltpu.VMEM((80,), jnp.int32),
            pltpu.SemaphoreType.DMA,
        ],
    )
    return f(y, srcp, dstlp, meta, zacc).reshape(NNP, DD)


def _degree16(srclp, meta, z16):
    def body(srclp_hbm, meta_hbm, z_hbm, out_hbm, idx_v, acc_v, meta_v):
        w = lax.axis_index("c") * NSUB + lax.axis_index("s")
        pltpu.sync_copy(meta_hbm, meta_v)
        pltpu.sync_copy(z_hbm, acc_v)
        off_w = meta_v[pl.ds(w, 16)][0]
        nch_w = meta_v[pl.ds(32 + w, 16)][0]
        ones = jnp.ones((16,), jnp.float32)

        def chunk(i, carry):
            off = (off_w + i) * CH
            pltpu.sync_copy(srclp_hbm.at[pl.ds(off, CH)],
                            idx_v.at[pl.ds(0, CH)])

            def edge(e, c2):
                base = idx_v[pl.ds(e, 16)][0] * 16
                plsc.addupdate(acc_v.at[pl.ds(base, 16)], ones)
                return c2

            lax.fori_loop(0, CH, edge, 0)
            return carry

        lax.fori_loop(0, nch_w, chunk, 0)
        pltpu.sync_copy(acc_v.at[pl.ds(0, NPT * 16)],
                        out_hbm.at[pl.ds(w * NPT * 16, NPT * 16)])

    f = pl.kernel(
        body,
        out_type=jax.ShapeDtypeStruct((NNP * 16,), jnp.float32),
        mesh=_sc_mesh(),
        scratch_types=[
            pltpu.VMEM((CH + 16,), jnp.int32),
            pltpu.VMEM((ACCR * 16,), jnp.float32),
            pltpu.VMEM((80,), jnp.int32),
        ],
    )
    return f(srclp, meta, z16).reshape(NNP, 16)


def _prep(deg16, x_all):
    tt = x_all.shape[0]

    def body(deg_ref, x_ref, dis_ref, y_ref):
        deg = jnp.sum(deg_ref[...], axis=1, keepdims=True) * (1.0 / 16.0)
        dis = jnp.where(deg > 0.0, lax.rsqrt(jnp.maximum(deg, 1e-12)), 0.0)
        dis_ref[...] = dis
        y_ref[0] = x_ref[0] * dis

    return pl.pallas_call(
        body,
        grid=(tt, NB),
        in_specs=[
            pl.BlockSpec((RB, 16), lambda t, i: (i, 0)),
            pl.BlockSpec((1, RB, DD), lambda t, i: (t, i, 0)),
        ],
        out_specs=[
            pl.BlockSpec((RB, 1), lambda t, i: (i, 0)),
            pl.BlockSpec((1, RB, DD), lambda t, i: (t, i, 0)),
        ],
        out_shape=[
            jax.ShapeDtypeStruct((NN, 1), jnp.float32),
            jax.ShapeDtypeStruct((tt, NN, DD), jnp.float32),
        ],
    )(deg16, x_all)


def _pmat(x, accx, dis, w0c, w1c, bc):
    def body(x_ref, a_ref, d_ref, w0_ref, w1_ref, b_ref, o_ref):
        agg = -(d_ref[...] * a_ref[...])
        p = jnp.dot(x_ref[...], w0_ref[...], preferred_element_type=jnp.float32)
        p += jnp.dot(agg, w1_ref[...], preferred_element_type=jnp.float32)
        o_ref[...] = p + b_ref[...]

    return pl.pallas_call(
        body,
        grid=(NB,),
        in_specs=[
            pl.BlockSpec((RB, DD), lambda i: (i, 0)),
            pl.BlockSpec((RB, DD), lambda i: (i, 0)),
            pl.BlockSpec((RB, 1), lambda i: (i, 0)),
            pl.BlockSpec((DD, 3 * DD), lambda i: (0, 0)),
            pl.BlockSpec((DD, 3 * DD), lambda i: (0, 0)),
            pl.BlockSpec((1, 3 * DD), lambda i: (0, 0)),
        ],
        out_specs=pl.BlockSpec((RB, 3 * DD), lambda i: (i, 0)),
        out_shape=jax.ShapeDtypeStruct((NN, 3 * DD), jnp.float32),
    )(x, accx, dis, w0c, w1c, bc)


def _step0(p0):
    def body(p_ref, h_ref):
        p = p_ref[...]
        z = jax.nn.sigmoid(p[:, :DD])
        ht = jnp.tanh(p[:, 2 * DD:])
        h_ref[...] = (1.0 - z) * ht

    return pl.pallas_call(
        body,
        grid=(NB,),
        in_specs=[pl.BlockSpec((RB, 3 * DD), lambda i: (i, 0))],
        out_specs=pl.BlockSpec((RB, DD), lambda i: (i, 0)),
        out_shape=jax.ShapeDtypeStruct((NN, DD), jnp.float32),
    )(p0)


def _scale(h, dis):
    def body(h_ref, d_ref, y_ref):
        y_ref[...] = h_ref[...] * d_ref[...]

    return pl.pallas_call(
        body,
        grid=(NB,),
        in_specs=[
            pl.BlockSpec((RB, DD), lambda i: (i, 0)),
            pl.BlockSpec((RB, 1), lambda i: (i, 0)),
        ],
        out_specs=pl.BlockSpec((RB, DD), lambda i: (i, 0)),
        out_shape=jax.ShapeDtypeStruct((NN, DD), jnp.float32),
    )(h, dis)


def _gates(p, h, acch, dis, wh0c, wh1c):
    def body(p_ref, h_ref, a_ref, d_ref, w0_ref, w1_ref,
             z_ref, hr_ref, yhr_ref):
        h = h_ref[...]
        dis = d_ref[...]
        agg = -(dis * a_ref[...])
        s = jnp.dot(h, w0_ref[...], preferred_element_type=jnp.float32)
        s += jnp.dot(agg, w1_ref[...], preferred_element_type=jnp.float32)
        s = jax.nn.sigmoid(p_ref[...] + s)
        z = s[:, :DD]
        r = s[:, DD:]
        hr = h * r
        z_ref[...] = z
        hr_ref[...] = hr
        yhr_ref[...] = dis * hr

    return pl.pallas_call(
        body,
        grid=(NB,),
        in_specs=[
            pl.BlockSpec((RB, 2 * DD), lambda i: (i, 0)),
            pl.BlockSpec((RB, DD), lambda i: (i, 0)),
            pl.BlockSpec((RB, DD), lambda i: (i, 0)),
            pl.BlockSpec((RB, 1), lambda i: (i, 0)),
            pl.BlockSpec((DD, 2 * DD), lambda i: (0, 0)),
            pl.BlockSpec((DD, 2 * DD), lambda i: (0, 0)),
        ],
        out_specs=[
            pl.BlockSpec((RB, DD), lambda i: (i, 0)),
            pl.BlockSpec((RB, DD), lambda i: (i, 0)),
            pl.BlockSpec((RB, DD), lambda i: (i, 0)),
        ],
        out_shape=[
            jax.ShapeDtypeStruct((NN, DD), jnp.float32),
            jax.ShapeDtypeStruct((NN, DD), jnp.float32),
            jax.ShapeDtypeStruct((NN, DD), jnp.float32),
        ],
    )(p, h, acch, dis, wh0c, wh1c)


def _update(ph, hr, acchr, dis, whh0, whh1, z, h):
    def body(p_ref, hr_ref, a_ref, d_ref, w0_ref, w1_ref, z_ref, h_ref,
             o_ref):
        agg = -(d_ref[...] * a_ref[...])
        s = jnp.dot(hr_ref[...], w0_ref[...],
                    preferred_element_type=jnp.float32)
        s += jnp.dot(agg, w1_ref[...], preferred_element_type=jnp.float32)
        ht = jnp.tanh(p_ref[...] + s)
        z = z_ref[...]
        o_ref[...] = z * h_ref[...] + (1.0 - z) * ht

    return pl.pallas_call(
        body,
        grid=(NB,),
        in_specs=[
            pl.BlockSpec((RB, DD), lambda i: (i, 2)),
            pl.BlockSpec((RB, DD), lambda i: (i, 0)),
            pl.BlockSpec((RB, DD), lambda i: (i, 0)),
            pl.BlockSpec((RB, 1), lambda i: (i, 0)),
            pl.BlockSpec((DD, DD), lambda i: (0, 0)),
            pl.BlockSpec((DD, DD), lambda i: (0, 0)),
            pl.BlockSpec((RB, DD), lambda i: (i, 0)),
            pl.BlockSpec((RB, DD), lambda i: (i, 0)),
        ],
        out_specs=pl.BlockSpec((RB, DD), lambda i: (i, 0)),
        out_shape=jax.ShapeDtypeStruct((NN, DD), jnp.float32),
    )(ph, hr, acchr, dis, whh0, whh1, z, h)


def _head(h, l1w, l1b, l2wt, l2b):
    def body(h_ref, w1_ref, b1_ref, w2_ref, b2_ref, o_ref):
        g = jnp.sum(h_ref[...], axis=0, keepdims=True) * (1.0 / NN)
        h1 = jax.nn.relu(
            jnp.dot(g, w1_ref[...], preferred_element_type=jnp.float32)
            + b1_ref[...])
        o_ref[...] = jnp.sum(h1 * w2_ref[...], axis=1,
                             keepdims=True) + b2_ref[...]

    return pl.pallas_call(
        body,
        grid=(1,),
        in_specs=[
            pl.BlockSpec((NN, DD), lambda i: (0, 0)),
            pl.BlockSpec((DD, DD), lambda i: (0, 0)),
            pl.BlockSpec((1, DD), lambda i: (0, 0)),
            pl.BlockSpec((1, DD), lambda i: (0, 0)),
            pl.BlockSpec((1, 1), lambda i: (0, 0)),
        ],
        out_specs=pl.BlockSpec((1, 1), lambda i: (0, 0)),
        out_shape=jax.ShapeDtypeStruct((1, 1), jnp.float32),
    )(h, l1w, l1b, l2wt, l2b)


def _partition32(key_idx, val_idx):
    order = jnp.argsort(key_idx)
    ks = key_idx[order]
    vs = val_idx[order]
    owner = ks // NPT
    cnt = jnp.zeros((NW,), jnp.int32).at[owner].add(1)
    padcnt = ((cnt + CH - 1) // CH) * CH
    zero1 = jnp.zeros((1,), jnp.int32)
    off = jnp.concatenate([zero1, jnp.cumsum(padcnt)[:-1]])
    start = jnp.concatenate([zero1, jnp.cumsum(cnt)[:-1]])
    pos = off[owner] + jnp.arange(EE, dtype=jnp.int32) - start[owner]
    varr = jnp.zeros((EPAD,), jnp.int32).at[pos].set(vs)
    karr = jnp.full((EPAD,), NPT, jnp.int32).at[pos].set(ks - owner * NPT)
    meta = jnp.zeros((80,), jnp.int32)
    meta = meta.at[jnp.arange(NW)].set(off // CH)
    meta = meta.at[NW + jnp.arange(NW)].set(padcnt // CH)
    return varr, karr, meta


def kernel(X_prefix, edge_index, Wxz0, Wxz1, bxz, Whz0, Whz1, bhz,
           Wxr0, Wxr1, bxr, Whr0, Whr1, bhr, Wxh0, Wxh1, bxh,
           Whh0, Whh1, bhh, L1W, L1b, L2W, L2b):
    src = edge_index[0]
    dst = edge_index[1]
    tt = X_prefix.shape[0]

    srcp, dstlp, meta_d = _partition32(dst, src)
    _, srclp, meta_s = _partition32(src, src)

    zacc = jnp.zeros((ACCR * DD,), jnp.float32)
    z16 = jnp.zeros((ACCR * 16,), jnp.float32)

    deg16 = _degree16(srclp, meta_s, z16)
    dis, y_all = _prep(deg16, X_prefix)

    w0c = jnp.concatenate([Wxz0, Wxr0, Wxh0], axis=1)
    w1c = jnp.concatenate([Wxz1, Wxr1, Wxh1], axis=1)
    bc = jnp.reshape(jnp.concatenate([bxz + bhz, bxr + bhr, bxh + bhh]),
                     (1, 3 * DD))
    p_list = []
    for t in range(tt):
        accx = _spmm(y_all[t], srcp, dstlp, meta_d, zacc)
        p_list.append(_pmat(X_prefix[t], accx, dis, w0c, w1c, bc))

    wh0c = jnp.concatenate([Whz0, Whr0], axis=1)
    wh1c = jnp.concatenate([Whz1, Whr1], axis=1)
    h = _step0(p_list[0])
    for t in range(1, tt):
        yh = _scale(h, dis)
        acch = _spmm(yh, srcp, dstlp, meta_d, zacc)
        z, hr, yhr = _gates(p_list[t][:, :2 * DD], h, acch, dis, wh0c, wh1c)
        acchr = _spmm(yhr, srcp, dstlp, meta_d, zacc)
        h = _update(p_list[t], hr, acchr, dis, Whh0, Whh1, z, h)

    out = _head(h, L1W, jnp.reshape(L1b, (1, DD)),
                jnp.reshape(L2W, (1, DD)), jnp.reshape(L2b, (1, 1)))
    return jnp.reshape(out, ())


# speedup vs baseline: 2.9358x; 1.3682x over previous
import jax
import jax.numpy as jnp
from jax import lax
from jax.experimental import pallas as pl
from jax.experimental.pallas import tpu as pltpu
from jax.experimental.pallas import tpu_sc as plsc

NN = 10000
DD = 256
EE = 160000
NSUB = 16
NCORE = 2
NW = NSUB * NCORE
NPT = 313
NNP = NW * NPT
ACCR = 320
CH = 128
EPAD = EE + NW * CH
RB = 1000
NB = NN // RB


def _sc_mesh():
    return plsc.VectorSubcoreMesh(core_axis_name="c", subcore_axis_name="s")


def _spmm(y, srcp, dstlp, meta, zacc):
    def body(y_hbm, srcp_hbm, dstlp_hbm, meta_hbm, z_hbm, out_hbm,
             idx_v, dst_v, rows_v, acc_v, meta_v, sem):
        w = lax.axis_index("c") * NSUB + lax.axis_index("s")
        pltpu.sync_copy(meta_hbm, meta_v)
        pltpu.sync_copy(z_hbm, acc_v)
        off_w = meta_v[pl.ds(w, 16)][0]
        nch_w = meta_v[pl.ds(32 + w, 16)][0]

        def chunk(i, carry):
            off = (off_w + i) * CH
            pltpu.sync_copy(srcp_hbm.at[pl.ds(off, CH)], idx_v)
            pltpu.sync_copy(dstlp_hbm.at[pl.ds(off, CH)],
                            dst_v.at[pl.ds(0, CH)])
            pltpu.async_copy(y_hbm.at[idx_v], rows_v, sem).wait()

            def group(g, c2):
                d16 = dst_v[pl.ds(g * 16, 16)]
                bases = [d16[e] * DD for e in range(16)]
                for e in range(16):
                    row = g * 16 + e
                    vals = [rows_v[row, pl.ds(j * 16, 16)]
                            for j in range(DD // 16)]
                    for j in range(DD // 16):
                        plsc.addupdate(
                            acc_v.at[pl.ds(bases[e] + j * 16, 16)], vals[j])
                return c2

            lax.fori_loop(0, CH // 16, group, 0)
            return carry

        lax.fori_loop(0, nch_w, chunk, 0)
        pltpu.sync_copy(acc_v.at[pl.ds(0, NPT * DD)],
                        out_hbm.at[pl.ds(w * NPT * DD, NPT * DD)])

    f = pl.kernel(
        body,
        out_type=jax.ShapeDtypeStruct((NNP * DD,), jnp.float32),
        mesh=_sc_mesh(),
        scratch_types=[
            pltpu.VMEM((CH,), jnp.int32),
            pltpu.VMEM((CH + 16,), jnp.int32),
            pltpu.VMEM((CH, DD), jnp.float32),
            pltpu.VMEM((ACCR * DD,), jnp.float32),
            pltpu.VMEM((80,), jnp.int32),
            pltpu.SemaphoreType.DMA,
        ],
    )
    return f(y, srcp, dstlp, meta, zacc).reshape(NNP, DD)


def _degree16(srclp, meta, z16):
    def body(srclp_hbm, meta_hbm, z_hbm, out_hbm, idx_v, acc_v, meta_v):
        w = lax.axis_index("c") * NSUB + lax.axis_index("s")
        pltpu.sync_copy(meta_hbm, meta_v)
        pltpu.sync_copy(z_hbm, acc_v)
        off_w = meta_v[pl.ds(w, 16)][0]
        nch_w = meta_v[pl.ds(32 + w, 16)][0]
        ones = jnp.ones((16,), jnp.float32)

        def chunk(i, carry):
            off = (off_w + i) * CH
            pltpu.sync_copy(srclp_hbm.at[pl.ds(off, CH)],
                            idx_v.at[pl.ds(0, CH)])

            def edge(e, c2):
                base = idx_v[pl.ds(e, 16)][0] * 16
                plsc.addupdate(acc_v.at[pl.ds(base, 16)], ones)
                return c2

            lax.fori_loop(0, CH, edge, 0)
            return carry

        lax.fori_loop(0, nch_w, chunk, 0)
        pltpu.sync_copy(acc_v.at[pl.ds(0, NPT * 16)],
                        out_hbm.at[pl.ds(w * NPT * 16, NPT * 16)])

    f = pl.kernel(
        body,
        out_type=jax.ShapeDtypeStruct((NNP * 16,), jnp.float32),
        mesh=_sc_mesh(),
        scratch_types=[
            pltpu.VMEM((CH + 16,), jnp.int32),
            pltpu.VMEM((ACCR * 16,), jnp.float32),
            pltpu.VMEM((80,), jnp.int32),
        ],
    )
    return f(srclp, meta, z16).reshape(NNP, 16)


def _prep(deg16, x_all):
    tt = x_all.shape[0]

    def body(deg_ref, x_ref, dis_ref, y_ref):
        deg = jnp.sum(deg_ref[...], axis=1, keepdims=True) * (1.0 / 16.0)
        dis = jnp.where(deg > 0.0, lax.rsqrt(jnp.maximum(deg, 1e-12)), 0.0)
        dis_ref[...] = dis
        y_ref[0] = x_ref[0] * dis

    return pl.pallas_call(
        body,
        grid=(tt, NB),
        in_specs=[
            pl.BlockSpec((RB, 16), lambda t, i: (i, 0)),
            pl.BlockSpec((1, RB, DD), lambda t, i: (t, i, 0)),
        ],
        out_specs=[
            pl.BlockSpec((RB, 1), lambda t, i: (i, 0)),
            pl.BlockSpec((1, RB, DD), lambda t, i: (t, i, 0)),
        ],
        out_shape=[
            jax.ShapeDtypeStruct((NN, 1), jnp.float32),
            jax.ShapeDtypeStruct((tt, NN, DD), jnp.float32),
        ],
    )(deg16, x_all)


def _pmat(x, accx, dis, w0c, w1c, bc):
    def body(x_ref, a_ref, d_ref, w0_ref, w1_ref, b_ref, o_ref):
        agg = -(d_ref[...] * a_ref[...])
        p = jnp.dot(x_ref[...], w0_ref[...], preferred_element_type=jnp.float32)
        p += jnp.dot(agg, w1_ref[...], preferred_element_type=jnp.float32)
        o_ref[...] = p + b_ref[...]

    return pl.pallas_call(
        body,
        grid=(NB,),
        in_specs=[
            pl.BlockSpec((RB, DD), lambda i: (i, 0)),
            pl.BlockSpec((RB, DD), lambda i: (i, 0)),
            pl.BlockSpec((RB, 1), lambda i: (i, 0)),
            pl.BlockSpec((DD, 3 * DD), lambda i: (0, 0)),
            pl.BlockSpec((DD, 3 * DD), lambda i: (0, 0)),
            pl.BlockSpec((1, 3 * DD), lambda i: (0, 0)),
        ],
        out_specs=pl.BlockSpec((RB, 3 * DD), lambda i: (i, 0)),
        out_shape=jax.ShapeDtypeStruct((NN, 3 * DD), jnp.float32),
    )(x, accx, dis, w0c, w1c, bc)


def _step0(p0):
    def body(p_ref, h_ref):
        p = p_ref[...]
        z = jax.nn.sigmoid(p[:, :DD])
        ht = jnp.tanh(p[:, 2 * DD:])
        h_ref[...] = (1.0 - z) * ht

    return pl.pallas_call(
        body,
        grid=(NB,),
        in_specs=[pl.BlockSpec((RB, 3 * DD), lambda i: (i, 0))],
        out_specs=pl.BlockSpec((RB, DD), lambda i: (i, 0)),
        out_shape=jax.ShapeDtypeStruct((NN, DD), jnp.float32),
    )(p0)


def _scale(h, dis):
    def body(h_ref, d_ref, y_ref):
        y_ref[...] = h_ref[...] * d_ref[...]

    return pl.pallas_call(
        body,
        grid=(NB,),
        in_specs=[
            pl.BlockSpec((RB, DD), lambda i: (i, 0)),
            pl.BlockSpec((RB, 1), lambda i: (i, 0)),
        ],
        out_specs=pl.BlockSpec((RB, DD), lambda i: (i, 0)),
        out_shape=jax.ShapeDtypeStruct((NN, DD), jnp.float32),
    )(h, dis)


def _gates(p, h, acch, dis, wh0c, wh1c):
    def body(p_ref, h_ref, a_ref, d_ref, w0_ref, w1_ref,
             z_ref, hr_ref, yhr_ref):
        h = h_ref[...]
        dis = d_ref[...]
        agg = -(dis * a_ref[...])
        s = jnp.dot(h, w0_ref[...], preferred_element_type=jnp.float32)
        s += jnp.dot(agg, w1_ref[...], preferred_element_type=jnp.float32)
        s = jax.nn.sigmoid(p_ref[...] + s)
        z = s[:, :DD]
        r = s[:, DD:]
        hr = h * r
        z_ref[...] = z
        hr_ref[...] = hr
        yhr_ref[...] = dis * hr

    return pl.pallas_call(
        body,
        grid=(NB,),
        in_specs=[
            pl.BlockSpec((RB, 2 * DD), lambda i: (i, 0)),
            pl.BlockSpec((RB, DD), lambda i: (i, 0)),
            pl.BlockSpec((RB, DD), lambda i: (i, 0)),
            pl.BlockSpec((RB, 1), lambda i: (i, 0)),
            pl.BlockSpec((DD, 2 * DD), lambda i: (0, 0)),
            pl.BlockSpec((DD, 2 * DD), lambda i: (0, 0)),
        ],
        out_specs=[
            pl.BlockSpec((RB, DD), lambda i: (i, 0)),
            pl.BlockSpec((RB, DD), lambda i: (i, 0)),
            pl.BlockSpec((RB, DD), lambda i: (i, 0)),
        ],
        out_shape=[
            jax.ShapeDtypeStruct((NN, DD), jnp.float32),
            jax.ShapeDtypeStruct((NN, DD), jnp.float32),
            jax.ShapeDtypeStruct((NN, DD), jnp.float32),
        ],
    )(p, h, acch, dis, wh0c, wh1c)


def _update(ph, hr, acchr, dis, whh0, whh1, z, h):
    def body(p_ref, hr_ref, a_ref, d_ref, w0_ref, w1_ref, z_ref, h_ref,
             o_ref):
        agg = -(d_ref[...] * a_ref[...])
        s = jnp.dot(hr_ref[...], w0_ref[...],
                    preferred_element_type=jnp.float32)
        s += jnp.dot(agg, w1_ref[...], preferred_element_type=jnp.float32)
        ht = jnp.tanh(p_ref[...] + s)
        z = z_ref[...]
        o_ref[...] = z * h_ref[...] + (1.0 - z) * ht

    return pl.pallas_call(
        body,
        grid=(NB,),
        in_specs=[
            pl.BlockSpec((RB, DD), lambda i: (i, 2)),
            pl.BlockSpec((RB, DD), lambda i: (i, 0)),
            pl.BlockSpec((RB, DD), lambda i: (i, 0)),
            pl.BlockSpec((RB, 1), lambda i: (i, 0)),
            pl.BlockSpec((DD, DD), lambda i: (0, 0)),
            pl.BlockSpec((DD, DD), lambda i: (0, 0)),
            pl.BlockSpec((RB, DD), lambda i: (i, 0)),
            pl.BlockSpec((RB, DD), lambda i: (i, 0)),
        ],
        out_specs=pl.BlockSpec((RB, DD), lambda i: (i, 0)),
        out_shape=jax.ShapeDtypeStruct((NN, DD), jnp.float32),
    )(ph, hr, acchr, dis, whh0, whh1, z, h)


def _head(h, l1w, l1b, l2wt, l2b):
    def body(h_ref, w1_ref, b1_ref, w2_ref, b2_ref, o_ref):
        g = jnp.sum(h_ref[...], axis=0, keepdims=True) * (1.0 / NN)
        h1 = jax.nn.relu(
            jnp.dot(g, w1_ref[...], preferred_element_type=jnp.float32)
            + b1_ref[...])
        o_ref[...] = jnp.sum(h1 * w2_ref[...], axis=1,
                             keepdims=True) + b2_ref[...]

    return pl.pallas_call(
        body,
        grid=(1,),
        in_specs=[
            pl.BlockSpec((NN, DD), lambda i: (0, 0)),
            pl.BlockSpec((DD, DD), lambda i: (0, 0)),
            pl.BlockSpec((1, DD), lambda i: (0, 0)),
            pl.BlockSpec((1, DD), lambda i: (0, 0)),
            pl.BlockSpec((1, 1), lambda i: (0, 0)),
        ],
        out_specs=pl.BlockSpec((1, 1), lambda i: (0, 0)),
        out_shape=jax.ShapeDtypeStruct((1, 1), jnp.float32),
    )(h, l1w, l1b, l2wt, l2b)


def _partition32(key_idx, val_idx):
    order = jnp.argsort(key_idx)
    ks = key_idx[order]
    vs = val_idx[order]
    owner = ks // NPT
    cnt = jnp.zeros((NW,), jnp.int32).at[owner].add(1)
    padcnt = ((cnt + CH - 1) // CH) * CH
    zero1 = jnp.zeros((1,), jnp.int32)
    off = jnp.concatenate([zero1, jnp.cumsum(padcnt)[:-1]])
    start = jnp.concatenate([zero1, jnp.cumsum(cnt)[:-1]])
    pos = off[owner] + jnp.arange(EE, dtype=jnp.int32) - start[owner]
    varr = jnp.zeros((EPAD,), jnp.int32).at[pos].set(vs)
    karr = jnp.full((EPAD,), NPT, jnp.int32).at[pos].set(ks - owner * NPT)
    meta = jnp.zeros((80,), jnp.int32)
    meta = meta.at[jnp.arange(NW)].set(off // CH)
    meta = meta.at[NW + jnp.arange(NW)].set(padcnt // CH)
    return varr, karr, meta


def kernel(X_prefix, edge_index, Wxz0, Wxz1, bxz, Whz0, Whz1, bhz,
           Wxr0, Wxr1, bxr, Whr0, Whr1, bhr, Wxh0, Wxh1, bxh,
           Whh0, Whh1, bhh, L1W, L1b, L2W, L2b):
    src = edge_index[0]
    dst = edge_index[1]
    tt = X_prefix.shape[0]

    srcp, dstlp, meta_d = _partition32(dst, src)
    _, srclp, meta_s = _partition32(src, src)

    zacc = jnp.zeros((ACCR * DD,), jnp.float32)
    z16 = jnp.zeros((ACCR * 16,), jnp.float32)

    deg16 = _degree16(srclp, meta_s, z16)
    dis, y_all = _prep(deg16, X_prefix)

    w0c = jnp.concatenate([Wxz0, Wxr0, Wxh0], axis=1)
    w1c = jnp.concatenate([Wxz1, Wxr1, Wxh1], axis=1)
    bc = jnp.reshape(jnp.concatenate([bxz + bhz, bxr + bhr, bxh + bhh]),
                     (1, 3 * DD))
    p_list = []
    for t in range(tt):
        accx = _spmm(y_all[t], srcp, dstlp, meta_d, zacc)
        p_list.append(_pmat(X_prefix[t], accx, dis, w0c, w1c, bc))

    wh0c = jnp.concatenate([Whz0, Whr0], axis=1)
    wh1c = jnp.concatenate([Whz1, Whr1], axis=1)
    h = _step0(p_list[0])
    for t in range(1, tt):
        yh = _scale(h, dis)
        acch = _spmm(yh, srcp, dstlp, meta_d, zacc)
        z, hr, yhr = _gates(p_list[t][:, :2 * DD], h, acch, dis, wh0c, wh1c)
        acchr = _spmm(yhr, srcp, dstlp, meta_d, zacc)
        h = _update(p_list[t], hr, acchr, dis, Whh0, Whh1, z, h)

    out = _head(h, L1W, jnp.reshape(L1b, (1, DD)),
                jnp.reshape(L2W, (1, DD)), jnp.reshape(L2b, (1, 1)))
    return jnp.reshape(out, ())


# speedup vs baseline: 3.4272x; 1.1674x over previous
import jax
import jax.numpy as jnp
from jax import lax
from jax.experimental import pallas as pl
from jax.experimental.pallas import tpu as pltpu
from jax.experimental.pallas import tpu_sc as plsc

NN = 10000
DD = 256
EE = 160000
NSUB = 16
NCORE = 2
NW = NSUB * NCORE
NPT = 313
NNP = NW * NPT
ACCR = 320
CH = 64
EPAD = EE + NW * CH
RB = 1000
NB = NN // RB


def _sc_mesh():
    return plsc.VectorSubcoreMesh(core_axis_name="c", subcore_axis_name="s")


def _spmm(y, srcp, dstlp, meta, zacc):
    def body(y_hbm, srcp_hbm, dstlp_hbm, meta_hbm, z_hbm, out_hbm,
             idx0_v, idx1_v, dst0_v, dst1_v, rows0_v, rows1_v,
             acc_v, meta_v, sem0, sem1):
        w = lax.axis_index("c") * NSUB + lax.axis_index("s")
        idx_b = (idx0_v, idx1_v)
        dst_b = (dst0_v, dst1_v)
        rows_b = (rows0_v, rows1_v)
        sem_b = (sem0, sem1)
        pltpu.sync_copy(meta_hbm, meta_v)
        pltpu.sync_copy(z_hbm, acc_v)
        off_w = meta_v[pl.ds(w, 16)][0]
        nch_w = meta_v[pl.ds(32 + w, 16)][0]

        def prefetch(ci, b):
            off = (off_w + ci) * CH
            pltpu.sync_copy(srcp_hbm.at[pl.ds(off, CH)], idx_b[b])
            pltpu.sync_copy(dstlp_hbm.at[pl.ds(off, CH)],
                            dst_b[b].at[pl.ds(0, CH)])
            pltpu.async_copy(y_hbm.at[idx_b[b]], rows_b[b], sem_b[b])

        def accumulate(b):
            dst_v = dst_b[b]
            rows_v = rows_b[b]

            def group(g, c2):
                d16 = dst_v[pl.ds(g * 16, 16)]
                bases = [d16[e] * DD for e in range(16)]
                for e in range(16):
                    row = g * 16 + e
                    vals = [rows_v[row, pl.ds(j * 16, 16)]
                            for j in range(DD // 16)]
                    for j in range(DD // 16):
                        plsc.addupdate(
                            acc_v.at[pl.ds(bases[e] + j * 16, 16)], vals[j])
                return c2

            lax.fori_loop(0, CH // 16, group, 0)

        @pl.when(nch_w > 0)
        def _():
            prefetch(0, 0)

        def pair(p, carry):
            for b in (0, 1):
                ci = p * 2 + b

                @pl.when(ci < nch_w)
                def _():
                    pltpu.make_async_copy(
                        y_hbm.at[idx_b[b]], rows_b[b], sem_b[b]).wait()

                    @pl.when(ci + 1 < nch_w)
                    def _():
                        prefetch(ci + 1, 1 - b)

                    accumulate(b)

            return carry

        lax.fori_loop(0, (nch_w + 1) // 2, pair, 0)
        pltpu.sync_copy(acc_v.at[pl.ds(0, NPT * DD)],
                        out_hbm.at[pl.ds(w * NPT * DD, NPT * DD)])

    f = pl.kernel(
        body,
        out_type=jax.ShapeDtypeStruct((NNP * DD,), jnp.float32),
        mesh=_sc_mesh(),
        scratch_types=[
            pltpu.VMEM((CH,), jnp.int32),
            pltpu.VMEM((CH,), jnp.int32),
            pltpu.VMEM((CH + 16,), jnp.int32),
            pltpu.VMEM((CH + 16,), jnp.int32),
            pltpu.VMEM((CH, DD), jnp.float32),
            pltpu.VMEM((CH, DD), jnp.float32),
            pltpu.VMEM((ACCR * DD,), jnp.float32),
            pltpu.VMEM((80,), jnp.int32),
            pltpu.SemaphoreType.DMA,
            pltpu.SemaphoreType.DMA,
        ],
    )
    return f(y, srcp, dstlp, meta, zacc).reshape(NNP, DD)


def _degree16(srclp, meta, z16):
    def body(srclp_hbm, meta_hbm, z_hbm, out_hbm, idx_v, acc_v, meta_v):
        w = lax.axis_index("c") * NSUB + lax.axis_index("s")
        pltpu.sync_copy(meta_hbm, meta_v)
        pltpu.sync_copy(z_hbm, acc_v)
        off_w = meta_v[pl.ds(w, 16)][0]
        nch_w = meta_v[pl.ds(32 + w, 16)][0]
        ones = jnp.ones((16,), jnp.float32)

        def chunk(i, carry):
            off = (off_w + i) * CH
            pltpu.sync_copy(srclp_hbm.at[pl.ds(off, CH)],
                            idx_v.at[pl.ds(0, CH)])

            def edge(e, c2):
                base = idx_v[pl.ds(e, 16)][0] * 16
                plsc.addupdate(acc_v.at[pl.ds(base, 16)], ones)
                return c2

            lax.fori_loop(0, CH, edge, 0)
            return carry

        lax.fori_loop(0, nch_w, chunk, 0)
        pltpu.sync_copy(acc_v.at[pl.ds(0, NPT * 16)],
                        out_hbm.at[pl.ds(w * NPT * 16, NPT * 16)])

    f = pl.kernel(
        body,
        out_type=jax.ShapeDtypeStruct((NNP * 16,), jnp.float32),
        mesh=_sc_mesh(),
        scratch_types=[
            pltpu.VMEM((CH + 16,), jnp.int32),
            pltpu.VMEM((ACCR * 16,), jnp.float32),
            pltpu.VMEM((80,), jnp.int32),
        ],
    )
    return f(srclp, meta, z16).reshape(NNP, 16)


def _prep(deg16, x_all):
    tt = x_all.shape[0]

    def body(deg_ref, x_ref, dis_ref, y_ref):
        deg = jnp.sum(deg_ref[...], axis=1, keepdims=True) * (1.0 / 16.0)
        dis = jnp.where(deg > 0.0, lax.rsqrt(jnp.maximum(deg, 1e-12)), 0.0)
        dis_ref[...] = dis
        y_ref[0] = x_ref[0] * dis

    return pl.pallas_call(
        body,
        grid=(tt, NB),
        in_specs=[
            pl.BlockSpec((RB, 16), lambda t, i: (i, 0)),
            pl.BlockSpec((1, RB, DD), lambda t, i: (t, i, 0)),
        ],
        out_specs=[
            pl.BlockSpec((RB, 1), lambda t, i: (i, 0)),
            pl.BlockSpec((1, RB, DD), lambda t, i: (t, i, 0)),
        ],
        out_shape=[
            jax.ShapeDtypeStruct((NN, 1), jnp.float32),
            jax.ShapeDtypeStruct((tt, NN, DD), jnp.float32),
        ],
    )(deg16, x_all)


def _pmat(x, accx, dis, w0c, w1c, bc):
    def body(x_ref, a_ref, d_ref, w0_ref, w1_ref, b_ref, o_ref):
        agg = -(d_ref[...] * a_ref[...])
        p = jnp.dot(x_ref[...], w0_ref[...], preferred_element_type=jnp.float32)
        p += jnp.dot(agg, w1_ref[...], preferred_element_type=jnp.float32)
        o_ref[...] = p + b_ref[...]

    return pl.pallas_call(
        body,
        grid=(NB,),
        in_specs=[
            pl.BlockSpec((RB, DD), lambda i: (i, 0)),
            pl.BlockSpec((RB, DD), lambda i: (i, 0)),
            pl.BlockSpec((RB, 1), lambda i: (i, 0)),
            pl.BlockSpec((DD, 3 * DD), lambda i: (0, 0)),
            pl.BlockSpec((DD, 3 * DD), lambda i: (0, 0)),
            pl.BlockSpec((1, 3 * DD), lambda i: (0, 0)),
        ],
        out_specs=pl.BlockSpec((RB, 3 * DD), lambda i: (i, 0)),
        out_shape=jax.ShapeDtypeStruct((NN, 3 * DD), jnp.float32),
    )(x, accx, dis, w0c, w1c, bc)


def _step0(p0):
    def body(p_ref, h_ref):
        p = p_ref[...]
        z = jax.nn.sigmoid(p[:, :DD])
        ht = jnp.tanh(p[:, 2 * DD:])
        h_ref[...] = (1.0 - z) * ht

    return pl.pallas_call(
        body,
        grid=(NB,),
        in_specs=[pl.BlockSpec((RB, 3 * DD), lambda i: (i, 0))],
        out_specs=pl.BlockSpec((RB, DD), lambda i: (i, 0)),
        out_shape=jax.ShapeDtypeStruct((NN, DD), jnp.float32),
    )(p0)


def _scale(h, dis):
    def body(h_ref, d_ref, y_ref):
        y_ref[...] = h_ref[...] * d_ref[...]

    return pl.pallas_call(
        body,
        grid=(NB,),
        in_specs=[
            pl.BlockSpec((RB, DD), lambda i: (i, 0)),
            pl.BlockSpec((RB, 1), lambda i: (i, 0)),
        ],
        out_specs=pl.BlockSpec((RB, DD), lambda i: (i, 0)),
        out_shape=jax.ShapeDtypeStruct((NN, DD), jnp.float32),
    )(h, dis)


def _gates(p, h, acch, dis, wh0c, wh1c):
    def body(p_ref, h_ref, a_ref, d_ref, w0_ref, w1_ref,
             z_ref, hr_ref, yhr_ref):
        h = h_ref[...]
        dis = d_ref[...]
        agg = -(dis * a_ref[...])
        s = jnp.dot(h, w0_ref[...], preferred_element_type=jnp.float32)
        s += jnp.dot(agg, w1_ref[...], preferred_element_type=jnp.float32)
        s = jax.nn.sigmoid(p_ref[...] + s)
        z = s[:, :DD]
        r = s[:, DD:]
        hr = h * r
        z_ref[...] = z
        hr_ref[...] = hr
        yhr_ref[...] = dis * hr

    return pl.pallas_call(
        body,
        grid=(NB,),
        in_specs=[
            pl.BlockSpec((RB, 2 * DD), lambda i: (i, 0)),
            pl.BlockSpec((RB, DD), lambda i: (i, 0)),
            pl.BlockSpec((RB, DD), lambda i: (i, 0)),
            pl.BlockSpec((RB, 1), lambda i: (i, 0)),
            pl.BlockSpec((DD, 2 * DD), lambda i: (0, 0)),
            pl.BlockSpec((DD, 2 * DD), lambda i: (0, 0)),
        ],
        out_specs=[
            pl.BlockSpec((RB, DD), lambda i: (i, 0)),
            pl.BlockSpec((RB, DD), lambda i: (i, 0)),
            pl.BlockSpec((RB, DD), lambda i: (i, 0)),
        ],
        out_shape=[
            jax.ShapeDtypeStruct((NN, DD), jnp.float32),
            jax.ShapeDtypeStruct((NN, DD), jnp.float32),
            jax.ShapeDtypeStruct((NN, DD), jnp.float32),
        ],
    )(p, h, acch, dis, wh0c, wh1c)


def _update(ph, hr, acchr, dis, whh0, whh1, z, h):
    def body(p_ref, hr_ref, a_ref, d_ref, w0_ref, w1_ref, z_ref, h_ref,
             o_ref):
        agg = -(d_ref[...] * a_ref[...])
        s = jnp.dot(hr_ref[...], w0_ref[...],
                    preferred_element_type=jnp.float32)
        s += jnp.dot(agg, w1_ref[...], preferred_element_type=jnp.float32)
        ht = jnp.tanh(p_ref[...] + s)
        z = z_ref[...]
        o_ref[...] = z * h_ref[...] + (1.0 - z) * ht

    return pl.pallas_call(
        body,
        grid=(NB,),
        in_specs=[
            pl.BlockSpec((RB, DD), lambda i: (i, 2)),
            pl.BlockSpec((RB, DD), lambda i: (i, 0)),
            pl.BlockSpec((RB, DD), lambda i: (i, 0)),
            pl.BlockSpec((RB, 1), lambda i: (i, 0)),
            pl.BlockSpec((DD, DD), lambda i: (0, 0)),
            pl.BlockSpec((DD, DD), lambda i: (0, 0)),
            pl.BlockSpec((RB, DD), lambda i: (i, 0)),
            pl.BlockSpec((RB, DD), lambda i: (i, 0)),
        ],
        out_specs=pl.BlockSpec((RB, DD), lambda i: (i, 0)),
        out_shape=jax.ShapeDtypeStruct((NN, DD), jnp.float32),
    )(ph, hr, acchr, dis, whh0, whh1, z, h)


def _head(h, l1w, l1b, l2wt, l2b):
    def body(h_ref, w1_ref, b1_ref, w2_ref, b2_ref, o_ref):
        g = jnp.sum(h_ref[...], axis=0, keepdims=True) * (1.0 / NN)
        h1 = jax.nn.relu(
            jnp.dot(g, w1_ref[...], preferred_element_type=jnp.float32)
            + b1_ref[...])
        o_ref[...] = jnp.sum(h1 * w2_ref[...], axis=1,
                             keepdims=True) + b2_ref[...]

    return pl.pallas_call(
        body,
        grid=(1,),
        in_specs=[
            pl.BlockSpec((NN, DD), lambda i: (0, 0)),
            pl.BlockSpec((DD, DD), lambda i: (0, 0)),
            pl.BlockSpec((1, DD), lambda i: (0, 0)),
            pl.BlockSpec((1, DD), lambda i: (0, 0)),
            pl.BlockSpec((1, 1), lambda i: (0, 0)),
        ],
        out_specs=pl.BlockSpec((1, 1), lambda i: (0, 0)),
        out_shape=jax.ShapeDtypeStruct((1, 1), jnp.float32),
    )(h, l1w, l1b, l2wt, l2b)


def _partition32(key_idx, val_idx):
    order = jnp.argsort(key_idx)
    ks = key_idx[order]
    vs = val_idx[order]
    owner = ks // NPT
    cnt = jnp.zeros((NW,), jnp.int32).at[owner].add(1)
    padcnt = ((cnt + CH - 1) // CH) * CH
    zero1 = jnp.zeros((1,), jnp.int32)
    off = jnp.concatenate([zero1, jnp.cumsum(padcnt)[:-1]])
    start = jnp.concatenate([zero1, jnp.cumsum(cnt)[:-1]])
    pos = off[owner] + jnp.arange(EE, dtype=jnp.int32) - start[owner]
    varr = jnp.zeros((EPAD,), jnp.int32).at[pos].set(vs)
    karr = jnp.full((EPAD,), NPT, jnp.int32).at[pos].set(ks - owner * NPT)
    meta = jnp.zeros((80,), jnp.int32)
    meta = meta.at[jnp.arange(NW)].set(off // CH)
    meta = meta.at[NW + jnp.arange(NW)].set(padcnt // CH)
    return varr, karr, meta


def kernel(X_prefix, edge_index, Wxz0, Wxz1, bxz, Whz0, Whz1, bhz,
           Wxr0, Wxr1, bxr, Whr0, Whr1, bhr, Wxh0, Wxh1, bxh,
           Whh0, Whh1, bhh, L1W, L1b, L2W, L2b):
    src = edge_index[0]
    dst = edge_index[1]
    tt = X_prefix.shape[0]

    srcp, dstlp, meta_d = _partition32(dst, src)
    _, srclp, meta_s = _partition32(src, src)

    zacc = jnp.zeros((ACCR * DD,), jnp.float32)
    z16 = jnp.zeros((ACCR * 16,), jnp.float32)

    deg16 = _degree16(srclp, meta_s, z16)
    dis, y_all = _prep(deg16, X_prefix)

    w0c = jnp.concatenate([Wxz0, Wxr0, Wxh0], axis=1)
    w1c = jnp.concatenate([Wxz1, Wxr1, Wxh1], axis=1)
    bc = jnp.reshape(jnp.concatenate([bxz + bhz, bxr + bhr, bxh + bhh]),
                     (1, 3 * DD))
    p_list = []
    for t in range(tt):
        accx = _spmm(y_all[t], srcp, dstlp, meta_d, zacc)
        p_list.append(_pmat(X_prefix[t], accx, dis, w0c, w1c, bc))

    wh0c = jnp.concatenate([Whz0, Whr0], axis=1)
    wh1c = jnp.concatenate([Whz1, Whr1], axis=1)
    h = _step0(p_list[0])
    for t in range(1, tt):
        yh = _scale(h, dis)
        acch = _spmm(yh, srcp, dstlp, meta_d, zacc)
        z, hr, yhr = _gates(p_list[t][:, :2 * DD], h, acch, dis, wh0c, wh1c)
        acchr = _spmm(yhr, srcp, dstlp, meta_d, zacc)
        h = _update(p_list[t], hr, acchr, dis, Whh0, Whh1, z, h)

    out = _head(h, L1W, jnp.reshape(L1b, (1, DD)),
                jnp.reshape(L2W, (1, DD)), jnp.reshape(L2b, (1, 1)))
    return jnp.reshape(out, ())


# speedup vs baseline: 5.2153x; 1.5218x over previous
import jax
import jax.numpy as jnp
from jax import lax
from jax.experimental import pallas as pl
from jax.experimental.pallas import tpu as pltpu
from jax.experimental.pallas import tpu_sc as plsc

NN = 10000
DD = 256
EE = 160000
NSUB = 16
NCORE = 2
NW = NSUB * NCORE
NPT = 313
NNP = NW * NPT
ACCR = 320
CH = 64
EPAD = EE + CH
RB = 1000
NB = NN // RB


def _sc_mesh():
    return plsc.VectorSubcoreMesh(core_axis_name="c", subcore_axis_name="s")


def _spmm(y, srcp, dstlp, meta, zacc):
    def body(y_hbm, srcp_hbm, dstlp_hbm, meta_hbm, z_hbm, out_hbm,
             idx0_v, idx1_v, dst0_v, dst1_v, rows0_v, rows1_v,
             acc_v, meta_v, sem0, sem1):
        w = lax.axis_index("c") * NSUB + lax.axis_index("s")
        idx_b = (idx0_v, idx1_v)
        dst_b = (dst0_v, dst1_v)
        rows_b = (rows0_v, rows1_v)
        sem_b = (sem0, sem1)
        pltpu.sync_copy(meta_hbm, meta_v)
        pltpu.sync_copy(z_hbm, acc_v)
        off_w = meta_v[pl.ds(w, 16)][0]
        nch_w = meta_v[pl.ds(32 + w, 16)][0]

        def prefetch(ci, b):
            off = pl.multiple_of(off_w + ci * CH, 8)
            pltpu.sync_copy(srcp_hbm.at[pl.ds(off, CH)], idx_b[b])
            pltpu.sync_copy(dstlp_hbm.at[pl.ds(off, CH)],
                            dst_b[b].at[pl.ds(0, CH)])
            pltpu.async_copy(y_hbm.at[idx_b[b]], rows_b[b], sem_b[b])

        def accumulate(b):
            dst_v = dst_b[b]
            rows_v = rows_b[b]

            def group(g, c2):
                c16 = dst_v[pl.ds(g * 16, 16)]
                d16 = jnp.where((c16 >> 10) == w, c16 & 1023, NPT)
                bases = [d16[e] * DD for e in range(16)]
                for e in range(16):
                    row = g * 16 + e
                    vals = [rows_v[row, pl.ds(j * 16, 16)]
                            for j in range(DD // 16)]
                    for j in range(DD // 16):
                        plsc.addupdate(
                            acc_v.at[pl.ds(bases[e] + j * 16, 16)], vals[j])
                return c2

            lax.fori_loop(0, CH // 16, group, 0)

        @pl.when(nch_w > 0)
        def _():
            prefetch(0, 0)

        def pair(p, carry):
            for b in (0, 1):
                ci = p * 2 + b

                @pl.when(ci < nch_w)
                def _():
                    pltpu.make_async_copy(
                        y_hbm.at[idx_b[b]], rows_b[b], sem_b[b]).wait()

                    @pl.when(ci + 1 < nch_w)
                    def _():
                        prefetch(ci + 1, 1 - b)

                    accumulate(b)

            return carry

        lax.fori_loop(0, (nch_w + 1) // 2, pair, 0)
        pltpu.sync_copy(acc_v.at[pl.ds(0, NPT * DD)],
                        out_hbm.at[pl.ds(w * NPT * DD, NPT * DD)])

    f = pl.kernel(
        body,
        out_type=jax.ShapeDtypeStruct((NNP * DD,), jnp.float32),
        mesh=_sc_mesh(),
        scratch_types=[
            pltpu.VMEM((CH,), jnp.int32),
            pltpu.VMEM((CH,), jnp.int32),
            pltpu.VMEM((CH + 16,), jnp.int32),
            pltpu.VMEM((CH + 16,), jnp.int32),
            pltpu.VMEM((CH, DD), jnp.float32),
            pltpu.VMEM((CH, DD), jnp.float32),
            pltpu.VMEM((ACCR * DD,), jnp.float32),
            pltpu.VMEM((80,), jnp.int32),
            pltpu.SemaphoreType.DMA,
            pltpu.SemaphoreType.DMA,
        ],
    )
    return f(y, srcp, dstlp, meta, zacc).reshape(NNP, DD)


def _degree16(srclp, meta, z16):
    def body(srclp_hbm, meta_hbm, z_hbm, out_hbm, idx_v, acc_v, meta_v):
        w = lax.axis_index("c") * NSUB + lax.axis_index("s")
        pltpu.sync_copy(meta_hbm, meta_v)
        pltpu.sync_copy(z_hbm, acc_v)
        off_w = meta_v[pl.ds(w, 16)][0]
        nch_w = meta_v[pl.ds(32 + w, 16)][0]
        ones = jnp.ones((16,), jnp.float32)

        def chunk(i, carry):
            off = pl.multiple_of(off_w + i * CH, 8)
            pltpu.sync_copy(srclp_hbm.at[pl.ds(off, CH)],
                            idx_v.at[pl.ds(0, CH)])

            def group(g, c2):
                c16 = idx_v[pl.ds(g * 16, 16)]
                d16 = jnp.where((c16 >> 10) == w, c16 & 1023, NPT) * 16
                for e in range(16):
                    plsc.addupdate(acc_v.at[pl.ds(d16[e], 16)], ones)
                return c2

            lax.fori_loop(0, CH // 16, group, 0)
            return carry

        lax.fori_loop(0, nch_w, chunk, 0)
        pltpu.sync_copy(acc_v.at[pl.ds(0, NPT * 16)],
                        out_hbm.at[pl.ds(w * NPT * 16, NPT * 16)])

    f = pl.kernel(
        body,
        out_type=jax.ShapeDtypeStruct((NNP * 16,), jnp.float32),
        mesh=_sc_mesh(),
        scratch_types=[
            pltpu.VMEM((CH + 16,), jnp.int32),
            pltpu.VMEM((ACCR * 16,), jnp.float32),
            pltpu.VMEM((80,), jnp.int32),
        ],
    )
    return f(srclp, meta, z16).reshape(NNP, 16)


def _prep(deg16, x_all):
    tt = x_all.shape[0]

    def body(deg_ref, x_ref, dis_ref, y_ref):
        deg = jnp.sum(deg_ref[...], axis=1, keepdims=True) * (1.0 / 16.0)
        dis = jnp.where(deg > 0.0, lax.rsqrt(jnp.maximum(deg, 1e-12)), 0.0)
        dis_ref[...] = dis
        y_ref[0] = x_ref[0] * dis

    return pl.pallas_call(
        body,
        grid=(tt, NB),
        in_specs=[
            pl.BlockSpec((RB, 16), lambda t, i: (i, 0)),
            pl.BlockSpec((1, RB, DD), lambda t, i: (t, i, 0)),
        ],
        out_specs=[
            pl.BlockSpec((RB, 1), lambda t, i: (i, 0)),
            pl.BlockSpec((1, RB, DD), lambda t, i: (t, i, 0)),
        ],
        out_shape=[
            jax.ShapeDtypeStruct((NN, 1), jnp.float32),
            jax.ShapeDtypeStruct((tt, NN, DD), jnp.float32),
        ],
    )(deg16, x_all)


def _pmat(x, accx, dis, w0c, w1c, bc):
    def body(x_ref, a_ref, d_ref, w0_ref, w1_ref, b_ref, o_ref):
        agg = -(d_ref[...] * a_ref[...])
        p = jnp.dot(x_ref[...], w0_ref[...], preferred_element_type=jnp.float32)
        p += jnp.dot(agg, w1_ref[...], preferred_element_type=jnp.float32)
        o_ref[...] = p + b_ref[...]

    return pl.pallas_call(
        body,
        grid=(NB,),
        in_specs=[
            pl.BlockSpec((RB, DD), lambda i: (i, 0)),
            pl.BlockSpec((RB, DD), lambda i: (i, 0)),
            pl.BlockSpec((RB, 1), lambda i: (i, 0)),
            pl.BlockSpec((DD, 3 * DD), lambda i: (0, 0)),
            pl.BlockSpec((DD, 3 * DD), lambda i: (0, 0)),
            pl.BlockSpec((1, 3 * DD), lambda i: (0, 0)),
        ],
        out_specs=pl.BlockSpec((RB, 3 * DD), lambda i: (i, 0)),
        out_shape=jax.ShapeDtypeStruct((NN, 3 * DD), jnp.float32),
    )(x, accx, dis, w0c, w1c, bc)


def _step0(p0):
    def body(p_ref, h_ref):
        p = p_ref[...]
        z = jax.nn.sigmoid(p[:, :DD])
        ht = jnp.tanh(p[:, 2 * DD:])
        h_ref[...] = (1.0 - z) * ht

    return pl.pallas_call(
        body,
        grid=(NB,),
        in_specs=[pl.BlockSpec((RB, 3 * DD), lambda i: (i, 0))],
        out_specs=pl.BlockSpec((RB, DD), lambda i: (i, 0)),
        out_shape=jax.ShapeDtypeStruct((NN, DD), jnp.float32),
    )(p0)


def _scale(h, dis):
    def body(h_ref, d_ref, y_ref):
        y_ref[...] = h_ref[...] * d_ref[...]

    return pl.pallas_call(
        body,
        grid=(NB,),
        in_specs=[
            pl.BlockSpec((RB, DD), lambda i: (i, 0)),
            pl.BlockSpec((RB, 1), lambda i: (i, 0)),
        ],
        out_specs=pl.BlockSpec((RB, DD), lambda i: (i, 0)),
        out_shape=jax.ShapeDtypeStruct((NN, DD), jnp.float32),
    )(h, dis)


def _gates(p, h, acch, dis, wh0c, wh1c):
    def body(p_ref, h_ref, a_ref, d_ref, w0_ref, w1_ref,
             z_ref, hr_ref, yhr_ref):
        h = h_ref[...]
        dis = d_ref[...]
        agg = -(dis * a_ref[...])
        s = jnp.dot(h, w0_ref[...], preferred_element_type=jnp.float32)
        s += jnp.dot(agg, w1_ref[...], preferred_element_type=jnp.float32)
        s = jax.nn.sigmoid(p_ref[...] + s)
        z = s[:, :DD]
        r = s[:, DD:]
        hr = h * r
        z_ref[...] = z
        hr_ref[...] = hr
        yhr_ref[...] = dis * hr

    return pl.pallas_call(
        body,
        grid=(NB,),
        in_specs=[
            pl.BlockSpec((RB, 2 * DD), lambda i: (i, 0)),
            pl.BlockSpec((RB, DD), lambda i: (i, 0)),
            pl.BlockSpec((RB, DD), lambda i: (i, 0)),
            pl.BlockSpec((RB, 1), lambda i: (i, 0)),
            pl.BlockSpec((DD, 2 * DD), lambda i: (0, 0)),
            pl.BlockSpec((DD, 2 * DD), lambda i: (0, 0)),
        ],
        out_specs=[
            pl.BlockSpec((RB, DD), lambda i: (i, 0)),
            pl.BlockSpec((RB, DD), lambda i: (i, 0)),
            pl.BlockSpec((RB, DD), lambda i: (i, 0)),
        ],
        out_shape=[
            jax.ShapeDtypeStruct((NN, DD), jnp.float32),
            jax.ShapeDtypeStruct((NN, DD), jnp.float32),
            jax.ShapeDtypeStruct((NN, DD), jnp.float32),
        ],
    )(p, h, acch, dis, wh0c, wh1c)


def _update(ph, hr, acchr, dis, whh0, whh1, z, h):
    def body(p_ref, hr_ref, a_ref, d_ref, w0_ref, w1_ref, z_ref, h_ref,
             o_ref):
        agg = -(d_ref[...] * a_ref[...])
        s = jnp.dot(hr_ref[...], w0_ref[...],
                    preferred_element_type=jnp.float32)
        s += jnp.dot(agg, w1_ref[...], preferred_element_type=jnp.float32)
        ht = jnp.tanh(p_ref[...] + s)
        z = z_ref[...]
        o_ref[...] = z * h_ref[...] + (1.0 - z) * ht

    return pl.pallas_call(
        body,
        grid=(NB,),
        in_specs=[
            pl.BlockSpec((RB, DD), lambda i: (i, 2)),
            pl.BlockSpec((RB, DD), lambda i: (i, 0)),
            pl.BlockSpec((RB, DD), lambda i: (i, 0)),
            pl.BlockSpec((RB, 1), lambda i: (i, 0)),
            pl.BlockSpec((DD, DD), lambda i: (0, 0)),
            pl.BlockSpec((DD, DD), lambda i: (0, 0)),
            pl.BlockSpec((RB, DD), lambda i: (i, 0)),
            pl.BlockSpec((RB, DD), lambda i: (i, 0)),
        ],
        out_specs=pl.BlockSpec((RB, DD), lambda i: (i, 0)),
        out_shape=jax.ShapeDtypeStruct((NN, DD), jnp.float32),
    )(ph, hr, acchr, dis, whh0, whh1, z, h)


def _head(h, l1w, l1b, l2wt, l2b):
    def body(h_ref, w1_ref, b1_ref, w2_ref, b2_ref, o_ref):
        g = jnp.sum(h_ref[...], axis=0, keepdims=True) * (1.0 / NN)
        h1 = jax.nn.relu(
            jnp.dot(g, w1_ref[...], preferred_element_type=jnp.float32)
            + b1_ref[...])
        o_ref[...] = jnp.sum(h1 * w2_ref[...], axis=1,
                             keepdims=True) + b2_ref[...]

    return pl.pallas_call(
        body,
        grid=(1,),
        in_specs=[
            pl.BlockSpec((NN, DD), lambda i: (0, 0)),
            pl.BlockSpec((DD, DD), lambda i: (0, 0)),
            pl.BlockSpec((1, DD), lambda i: (0, 0)),
            pl.BlockSpec((1, DD), lambda i: (0, 0)),
            pl.BlockSpec((1, 1), lambda i: (0, 0)),
        ],
        out_specs=pl.BlockSpec((1, 1), lambda i: (0, 0)),
        out_shape=jax.ShapeDtypeStruct((1, 1), jnp.float32),
    )(h, l1w, l1b, l2wt, l2b)


def _edge_groups(key_idx, val_idx):
    order = jnp.argsort(key_idx)
    ks = key_idx[order]
    vs = val_idx[order]
    owner = ks // NPT
    code = owner * 1024 + (ks - owner * NPT)
    varr = jnp.concatenate([vs, jnp.zeros((CH,), jnp.int32)])
    karr = jnp.concatenate(
        [code, jnp.full((CH,), NW * 1024 + NPT, jnp.int32)])
    tile0 = jnp.arange(NW, dtype=jnp.int32) * NPT
    start = jnp.searchsorted(ks, tile0).astype(jnp.int32)
    end = jnp.concatenate([start[1:], jnp.full((1,), EE, jnp.int32)])
    astart = start & ~7
    nch = (end - astart + CH - 1) // CH
    meta = jnp.zeros((80,), jnp.int32)
    meta = meta.at[jnp.arange(NW)].set(astart)
    meta = meta.at[NW + jnp.arange(NW)].set(nch)
    return varr, karr, meta


def kernel(X_prefix, edge_index, Wxz0, Wxz1, bxz, Whz0, Whz1, bhz,
           Wxr0, Wxr1, bxr, Whr0, Whr1, bhr, Wxh0, Wxh1, bxh,
           Whh0, Whh1, bhh, L1W, L1b, L2W, L2b):
    src = edge_index[0]
    dst = edge_index[1]
    tt = X_prefix.shape[0]

    srcp, dstlp, meta_d = _edge_groups(dst, src)
    _, srclp, meta_s = _edge_groups(src, src)

    zacc = jnp.zeros((ACCR * DD,), jnp.float32)
    z16 = jnp.zeros((ACCR * 16,), jnp.float32)

    deg16 = _degree16(srclp, meta_s, z16)
    dis, y_all = _prep(deg16, X_prefix)

    w0c = jnp.concatenate([Wxz0, Wxr0, Wxh0], axis=1)
    w1c = jnp.concatenate([Wxz1, Wxr1, Wxh1], axis=1)
    bc = jnp.reshape(jnp.concatenate([bxz + bhz, bxr + bhr, bxh + bhh]),
                     (1, 3 * DD))
    p_list = []
    for t in range(tt):
        accx = _spmm(y_all[t], srcp, dstlp, meta_d, zacc)
        p_list.append(_pmat(X_prefix[t], accx, dis, w0c, w1c, bc))

    wh0c = jnp.concatenate([Whz0, Whr0], axis=1)
    wh1c = jnp.concatenate([Whz1, Whr1], axis=1)
    h = _step0(p_list[0])
    for t in range(1, tt):
        yh = _scale(h, dis)
        acch = _spmm(yh, srcp, dstlp, meta_d, zacc)
        z, hr, yhr = _gates(p_list[t][:, :2 * DD], h, acch, dis, wh0c, wh1c)
        acchr = _spmm(yhr, srcp, dstlp, meta_d, zacc)
        h = _update(p_list[t], hr, acchr, dis, Whh0, Whh1, z, h)

    out = _head(h, L1W, jnp.reshape(L1b, (1, DD)),
                jnp.reshape(L2W, (1, DD)), jnp.reshape(L2b, (1, 1)))
    return jnp.reshape(out, ())


# speedup vs baseline: 5.2336x; 1.0035x over previous
import jax
import jax.numpy as jnp
from jax import lax
from jax.experimental import pallas as pl
from jax.experimental.pallas import tpu as pltpu
from jax.experimental.pallas import tpu_sc as plsc

NN = 10000
DD = 256
EE = 160000
NSUB = 16
NCORE = 2
NW = NSUB * NCORE
NPT = 313
NNP = NW * NPT
ACCR = 320
CH = 64
EPAD = EE + CH
RB = 1000
NB = NN // RB


def _sc_mesh():
    return plsc.VectorSubcoreMesh(core_axis_name="c", subcore_axis_name="s")


def _spmm(y, srcp, dstlp, meta, zacc):
    def body(y_hbm, srcp_hbm, dstlp_hbm, meta_hbm, z_hbm, out_hbm,
             idx0_v, idx1_v, dst0_v, dst1_v, rows0_v, rows1_v,
             acc_v, meta_v, sem0, sem1):
        w = lax.axis_index("c") * NSUB + lax.axis_index("s")
        idx_b = (idx0_v, idx1_v)
        dst_b = (dst0_v, dst1_v)
        rows_b = (rows0_v, rows1_v)
        sem_b = (sem0, sem1)
        pltpu.sync_copy(meta_hbm, meta_v)
        pltpu.sync_copy(z_hbm, acc_v)
        off_w = meta_v[pl.ds(w, 16)][0]
        nch_w = meta_v[pl.ds(32 + w, 16)][0]

        def prefetch(ci, b):
            off = pl.multiple_of(off_w + ci * CH, 8)
            pltpu.sync_copy(srcp_hbm.at[pl.ds(off, CH)], idx_b[b])
            pltpu.sync_copy(dstlp_hbm.at[pl.ds(off, CH)],
                            dst_b[b].at[pl.ds(0, CH)])
            pltpu.async_copy(y_hbm.at[idx_b[b]], rows_b[b], sem_b[b])

        def accumulate(b):
            dst_v = dst_b[b]
            rows_v = rows_b[b]

            def group(g, c2):
                c16 = dst_v[pl.ds(g * 16, 16)]
                d16 = jnp.where((c16 >> 10) == w, c16 & 1023, NPT)
                bases = [d16[e] * DD for e in range(16)]
                for e in range(16):
                    row = g * 16 + e
                    vals = [rows_v[row, pl.ds(j * 16, 16)]
                            for j in range(DD // 16)]
                    for j in range(DD // 16):
                        plsc.addupdate(
                            acc_v.at[pl.ds(bases[e] + j * 16, 16)], vals[j])
                return c2

            lax.fori_loop(0, CH // 16, group, 0)

        @pl.when(nch_w > 0)
        def _():
            prefetch(0, 0)

        def pair(p, carry):
            for b in (0, 1):
                ci = p * 2 + b

                @pl.when(ci < nch_w)
                def _():
                    pltpu.make_async_copy(
                        y_hbm.at[idx_b[b]], rows_b[b], sem_b[b]).wait()

                    @pl.when(ci + 1 < nch_w)
                    def _():
                        prefetch(ci + 1, 1 - b)

                    accumulate(b)

            return carry

        lax.fori_loop(0, (nch_w + 1) // 2, pair, 0)
        pltpu.sync_copy(acc_v.at[pl.ds(0, NPT * DD)],
                        out_hbm.at[pl.ds(w * NPT * DD, NPT * DD)])

    f = pl.kernel(
        body,
        out_type=jax.ShapeDtypeStruct((NNP * DD,), jnp.float32),
        mesh=_sc_mesh(),
        scratch_types=[
            pltpu.VMEM((CH,), jnp.int32),
            pltpu.VMEM((CH,), jnp.int32),
            pltpu.VMEM((CH + 16,), jnp.int32),
            pltpu.VMEM((CH + 16,), jnp.int32),
            pltpu.VMEM((CH, DD), jnp.float32),
            pltpu.VMEM((CH, DD), jnp.float32),
            pltpu.VMEM((ACCR * DD,), jnp.float32),
            pltpu.VMEM((80,), jnp.int32),
            pltpu.SemaphoreType.DMA,
            pltpu.SemaphoreType.DMA,
        ],
    )
    return f(y, srcp, dstlp, meta, zacc).reshape(NNP, DD)


def _degree16(srclp, meta, z16):
    def body(srclp_hbm, meta_hbm, z_hbm, out_hbm, idx_v, acc_v, meta_v):
        w = lax.axis_index("c") * NSUB + lax.axis_index("s")
        pltpu.sync_copy(meta_hbm, meta_v)
        pltpu.sync_copy(z_hbm, acc_v)
        off_w = meta_v[pl.ds(w, 16)][0]
        nch_w = meta_v[pl.ds(32 + w, 16)][0]
        ones = jnp.ones((16,), jnp.float32)

        def chunk(i, carry):
            off = pl.multiple_of(off_w + i * CH, 8)
            pltpu.sync_copy(srclp_hbm.at[pl.ds(off, CH)],
                            idx_v.at[pl.ds(0, CH)])

            def group(g, c2):
                c16 = idx_v[pl.ds(g * 16, 16)]
                d16 = jnp.where((c16 >> 10) == w, c16 & 1023, NPT) * 16
                for e in range(16):
                    plsc.addupdate(acc_v.at[pl.ds(d16[e], 16)], ones)
                return c2

            lax.fori_loop(0, CH // 16, group, 0)
            return carry

        lax.fori_loop(0, nch_w, chunk, 0)
        pltpu.sync_copy(acc_v.at[pl.ds(0, NPT * 16)],
                        out_hbm.at[pl.ds(w * NPT * 16, NPT * 16)])

    f = pl.kernel(
        body,
        out_type=jax.ShapeDtypeStruct((NNP * 16,), jnp.float32),
        mesh=_sc_mesh(),
        scratch_types=[
            pltpu.VMEM((CH + 16,), jnp.int32),
            pltpu.VMEM((ACCR * 16,), jnp.float32),
            pltpu.VMEM((80,), jnp.int32),
        ],
    )
    return f(srclp, meta, z16).reshape(NNP, 16)


def _prep(deg16, x_all):
    tt = x_all.shape[0]

    def body(deg_ref, x_ref, dis_ref, y_ref):
        deg = jnp.sum(deg_ref[...], axis=1, keepdims=True) * (1.0 / 16.0)
        dis = jnp.where(deg > 0.0, lax.rsqrt(jnp.maximum(deg, 1e-12)), 0.0)
        dis_ref[...] = dis
        y_ref[0] = x_ref[0] * dis

    return pl.pallas_call(
        body,
        grid=(tt, NB),
        in_specs=[
            pl.BlockSpec((RB, 16), lambda t, i: (i, 0)),
            pl.BlockSpec((1, RB, DD), lambda t, i: (t, i, 0)),
        ],
        out_specs=[
            pl.BlockSpec((RB, 1), lambda t, i: (i, 0)),
            pl.BlockSpec((1, RB, DD), lambda t, i: (t, i, 0)),
        ],
        out_shape=[
            jax.ShapeDtypeStruct((NN, 1), jnp.float32),
            jax.ShapeDtypeStruct((tt, NN, DD), jnp.float32),
        ],
    )(deg16, x_all)


def _pmat(x, accx, dis, w0c, w1c, bc):
    def body(x_ref, a_ref, d_ref, w0_ref, w1_ref, b_ref, o_ref):
        agg = -(d_ref[...] * a_ref[...])
        p = jnp.dot(x_ref[...], w0_ref[...], preferred_element_type=jnp.float32)
        p += jnp.dot(agg, w1_ref[...], preferred_element_type=jnp.float32)
        o_ref[...] = p + b_ref[...]

    return pl.pallas_call(
        body,
        grid=(NB,),
        in_specs=[
            pl.BlockSpec((RB, DD), lambda i: (i, 0)),
            pl.BlockSpec((RB, DD), lambda i: (i, 0)),
            pl.BlockSpec((RB, 1), lambda i: (i, 0)),
            pl.BlockSpec((DD, 3 * DD), lambda i: (0, 0)),
            pl.BlockSpec((DD, 3 * DD), lambda i: (0, 0)),
            pl.BlockSpec((1, 3 * DD), lambda i: (0, 0)),
        ],
        out_specs=pl.BlockSpec((RB, 3 * DD), lambda i: (i, 0)),
        out_shape=jax.ShapeDtypeStruct((NN, 3 * DD), jnp.float32),
    )(x, accx, dis, w0c, w1c, bc)


def _step0(p0, dis):
    def body(p_ref, d_ref, h_ref, yh_ref):
        p = p_ref[...]
        z = jax.nn.sigmoid(p[:, :DD])
        ht = jnp.tanh(p[:, 2 * DD:])
        h = (1.0 - z) * ht
        h_ref[...] = h
        yh_ref[...] = d_ref[...] * h

    return pl.pallas_call(
        body,
        grid=(NB,),
        in_specs=[
            pl.BlockSpec((RB, 3 * DD), lambda i: (i, 0)),
            pl.BlockSpec((RB, 1), lambda i: (i, 0)),
        ],
        out_specs=[
            pl.BlockSpec((RB, DD), lambda i: (i, 0)),
            pl.BlockSpec((RB, DD), lambda i: (i, 0)),
        ],
        out_shape=[
            jax.ShapeDtypeStruct((NN, DD), jnp.float32),
            jax.ShapeDtypeStruct((NN, DD), jnp.float32),
        ],
    )(p0, dis)


def _scale(h, dis):
    def body(h_ref, d_ref, y_ref):
        y_ref[...] = h_ref[...] * d_ref[...]

    return pl.pallas_call(
        body,
        grid=(NB,),
        in_specs=[
            pl.BlockSpec((RB, DD), lambda i: (i, 0)),
            pl.BlockSpec((RB, 1), lambda i: (i, 0)),
        ],
        out_specs=pl.BlockSpec((RB, DD), lambda i: (i, 0)),
        out_shape=jax.ShapeDtypeStruct((NN, DD), jnp.float32),
    )(h, dis)


def _gates(p, h, acch, dis, wh0c, wh1c):
    def body(p_ref, h_ref, a_ref, d_ref, w0_ref, w1_ref,
             z_ref, hr_ref, yhr_ref):
        h = h_ref[...]
        dis = d_ref[...]
        agg = -(dis * a_ref[...])
        s = jnp.dot(h, w0_ref[...], preferred_element_type=jnp.float32)
        s += jnp.dot(agg, w1_ref[...], preferred_element_type=jnp.float32)
        s = jax.nn.sigmoid(p_ref[...] + s)
        z = s[:, :DD]
        r = s[:, DD:]
        hr = h * r
        z_ref[...] = z
        hr_ref[...] = hr
        yhr_ref[...] = dis * hr

    return pl.pallas_call(
        body,
        grid=(NB,),
        in_specs=[
            pl.BlockSpec((RB, 2 * DD), lambda i: (i, 0)),
            pl.BlockSpec((RB, DD), lambda i: (i, 0)),
            pl.BlockSpec((RB, DD), lambda i: (i, 0)),
            pl.BlockSpec((RB, 1), lambda i: (i, 0)),
            pl.BlockSpec((DD, 2 * DD), lambda i: (0, 0)),
            pl.BlockSpec((DD, 2 * DD), lambda i: (0, 0)),
        ],
        out_specs=[
            pl.BlockSpec((RB, DD), lambda i: (i, 0)),
            pl.BlockSpec((RB, DD), lambda i: (i, 0)),
            pl.BlockSpec((RB, DD), lambda i: (i, 0)),
        ],
        out_shape=[
            jax.ShapeDtypeStruct((NN, DD), jnp.float32),
            jax.ShapeDtypeStruct((NN, DD), jnp.float32),
            jax.ShapeDtypeStruct((NN, DD), jnp.float32),
        ],
    )(p, h, acch, dis, wh0c, wh1c)


def _update(ph, hr, acchr, dis, whh0, whh1, z, h):
    def body(p_ref, hr_ref, a_ref, d_ref, w0_ref, w1_ref, z_ref, h_ref,
             o_ref, yh_ref):
        agg = -(d_ref[...] * a_ref[...])
        s = jnp.dot(hr_ref[...], w0_ref[...],
                    preferred_element_type=jnp.float32)
        s += jnp.dot(agg, w1_ref[...], preferred_element_type=jnp.float32)
        ht = jnp.tanh(p_ref[...] + s)
        z = z_ref[...]
        h2 = z * h_ref[...] + (1.0 - z) * ht
        o_ref[...] = h2
        yh_ref[...] = d_ref[...] * h2

    return pl.pallas_call(
        body,
        grid=(NB,),
        in_specs=[
            pl.BlockSpec((RB, DD), lambda i: (i, 2)),
            pl.BlockSpec((RB, DD), lambda i: (i, 0)),
            pl.BlockSpec((RB, DD), lambda i: (i, 0)),
            pl.BlockSpec((RB, 1), lambda i: (i, 0)),
            pl.BlockSpec((DD, DD), lambda i: (0, 0)),
            pl.BlockSpec((DD, DD), lambda i: (0, 0)),
            pl.BlockSpec((RB, DD), lambda i: (i, 0)),
            pl.BlockSpec((RB, DD), lambda i: (i, 0)),
        ],
        out_specs=[
            pl.BlockSpec((RB, DD), lambda i: (i, 0)),
            pl.BlockSpec((RB, DD), lambda i: (i, 0)),
        ],
        out_shape=[
            jax.ShapeDtypeStruct((NN, DD), jnp.float32),
            jax.ShapeDtypeStruct((NN, DD), jnp.float32),
        ],
    )(ph, hr, acchr, dis, whh0, whh1, z, h)


def _head(h, l1w, l1b, l2wt, l2b):
    def body(h_ref, w1_ref, b1_ref, w2_ref, b2_ref, o_ref):
        g = jnp.sum(h_ref[...], axis=0, keepdims=True) * (1.0 / NN)
        h1 = jax.nn.relu(
            jnp.dot(g, w1_ref[...], preferred_element_type=jnp.float32)
            + b1_ref[...])
        o_ref[...] = jnp.sum(h1 * w2_ref[...], axis=1,
                             keepdims=True) + b2_ref[...]

    return pl.pallas_call(
        body,
        grid=(1,),
        in_specs=[
            pl.BlockSpec((NN, DD), lambda i: (0, 0)),
            pl.BlockSpec((DD, DD), lambda i: (0, 0)),
            pl.BlockSpec((1, DD), lambda i: (0, 0)),
            pl.BlockSpec((1, DD), lambda i: (0, 0)),
            pl.BlockSpec((1, 1), lambda i: (0, 0)),
        ],
        out_specs=pl.BlockSpec((1, 1), lambda i: (0, 0)),
        out_shape=jax.ShapeDtypeStruct((1, 1), jnp.float32),
    )(h, l1w, l1b, l2wt, l2b)


def _edge_groups(key_idx, val_idx):
    order = jnp.argsort(key_idx)
    ks = key_idx[order]
    vs = val_idx[order]
    owner = ks // NPT
    code = owner * 1024 + (ks - owner * NPT)
    varr = jnp.concatenate([vs, jnp.zeros((CH,), jnp.int32)])
    karr = jnp.concatenate(
        [code, jnp.full((CH,), NW * 1024 + NPT, jnp.int32)])
    tile0 = jnp.arange(NW, dtype=jnp.int32) * NPT
    start = jnp.searchsorted(ks, tile0).astype(jnp.int32)
    end = jnp.concatenate([start[1:], jnp.full((1,), EE, jnp.int32)])
    astart = start & ~7
    nch = (end - astart + CH - 1) // CH
    meta = jnp.zeros((80,), jnp.int32)
    meta = meta.at[jnp.arange(NW)].set(astart)
    meta = meta.at[NW + jnp.arange(NW)].set(nch)
    return varr, karr, meta


def kernel(X_prefix, edge_index, Wxz0, Wxz1, bxz, Whz0, Whz1, bhz,
           Wxr0, Wxr1, bxr, Whr0, Whr1, bhr, Wxh0, Wxh1, bxh,
           Whh0, Whh1, bhh, L1W, L1b, L2W, L2b):
    src = edge_index[0]
    dst = edge_index[1]
    tt = X_prefix.shape[0]

    srcp, dstlp, meta_d = _edge_groups(dst, src)
    _, srclp, meta_s = _edge_groups(src, src)

    zacc = jnp.zeros((ACCR * DD,), jnp.float32)
    z16 = jnp.zeros((ACCR * 16,), jnp.float32)

    deg16 = _degree16(srclp, meta_s, z16)
    dis, y_all = _prep(deg16, X_prefix)

    w0c = jnp.concatenate([Wxz0, Wxr0, Wxh0], axis=1)
    w1c = jnp.concatenate([Wxz1, Wxr1, Wxh1], axis=1)
    bc = jnp.reshape(jnp.concatenate([bxz + bhz, bxr + bhr, bxh + bhh]),
                     (1, 3 * DD))
    p_list = []
    for t in range(tt):
        accx = _spmm(y_all[t], srcp, dstlp, meta_d, zacc)
        p_list.append(_pmat(X_prefix[t], accx, dis, w0c, w1c, bc))

    wh0c = jnp.concatenate([Whz0, Whr0], axis=1)
    wh1c = jnp.concatenate([Whz1, Whr1], axis=1)
    h, yh = _step0(p_list[0], dis)
    for t in range(1, tt):
        acch = _spmm(yh, srcp, dstlp, meta_d, zacc)
        z, hr, yhr = _gates(p_list[t][:, :2 * DD], h, acch, dis, wh0c, wh1c)
        acchr = _spmm(yhr, srcp, dstlp, meta_d, zacc)
        h, yh = _update(p_list[t], hr, acchr, dis, Whh0, Whh1, z, h)

    out = _head(h, L1W, jnp.reshape(L1b, (1, DD)),
                jnp.reshape(L2W, (1, DD)), jnp.reshape(L2b, (1, 1)))
    return jnp.reshape(out, ())
